# Initial kernel scaffold; baseline (speedup 1.0000x reference)
import functools

import jax
import jax.numpy as jnp
from jax import lax
from jax.experimental import pallas as pl
from jax.experimental.pallas import tpu as pltpu
from jax.experimental.pallas import tpu_sc as plsc

_LW = 8
_SC_CHUNK = 256
_NW = 32


def _embed_gather(x_flat, embed):
    ntok = x_flat.shape[0]
    d = embed.shape[1]
    bpw = ntok // _NW
    mesh = plsc.VectorSubcoreMesh(core_axis_name="c", subcore_axis_name="s")

    @functools.partial(
        pl.kernel,
        mesh=mesh,
        out_type=jax.ShapeDtypeStruct((ntok, d), jnp.float32),
        scratch_types=[
            pltpu.VMEM((bpw,), jnp.int32),
            pltpu.VMEM((bpw, d), jnp.float32),
            pltpu.SemaphoreType.DMA,
        ],
    )
    def gather_kernel(table_hbm, idx_hbm, out_hbm, idx_v, rows_v, sem):
        wid = lax.axis_index("s") * 2 + lax.axis_index("c")
        base = wid * bpw
        pltpu.sync_copy(idx_hbm.at[pl.ds(base, bpw)], idx_v)
        pltpu.async_copy(table_hbm.at[idx_v], rows_v, sem).wait()
        pltpu.sync_copy(rows_v, out_hbm.at[pl.ds(base, bpw)])

    return gather_kernel(embed, x_flat)


def _tc_body(h_ref, td_ref, wk_ref, wv_ref, wr_ref, mkt_ref, mv_ref,
             wq_ref, wo_ref, ow_ref, ob_ref, out_ref,
             kbuf, vbuf, wkvbuf, a_ref, b_ref):
    bsz = h_ref.shape[0]
    dmodel = h_ref.shape[2]
    vocab = out_ref.shape[2]
    cap = 50
    rows = bsz * _SC_CHUNK
    f32 = jnp.float32

    @pl.when(pl.program_id(0) == 0)
    def _init():
        a_ref[...] = jnp.zeros_like(a_ref)
        b_ref[...] = jnp.zeros_like(b_ref)

    hm = h_ref[...].reshape(rows, dmodel)
    k2 = jnp.dot(hm, wk_ref[...], preferred_element_type=f32)
    v2 = jnp.dot(hm, wv_ref[...], preferred_element_type=f32)
    r2 = 1.0 / (1.0 + jnp.exp(-jnp.dot(hm, wr_ref[...], preferred_element_type=f32)))
    kbuf[...] = k2.reshape(bsz, _SC_CHUNK, dmodel)
    vbuf[...] = v2.reshape(bsz, _SC_CHUNK, dmodel)

    neg_ld = jnp.exp(td_ref[...])
    tpow = lax.broadcasted_iota(f32, (_LW, dmodel), 0)
    dpow = jnp.exp(-tpow * neg_ld)[None]
    dinv = jnp.exp(tpow * neg_ld)[None]
    dnext = dpow * jnp.exp(-neg_ld)[None]

    def win(j, carry):
        a, b = carry
        kk = kbuf[:, pl.ds(j * _LW, _LW), :]
        vv = vbuf[:, pl.ds(j * _LW, _LW), :]
        ek = jnp.exp(jnp.clip(kk, -30.0, 30.0))
        ua = ek * vv * dinv
        ub = ek * dinv
        for sh in (1, 2, 4):
            z = jnp.zeros((bsz, sh, dmodel), f32)
            ua = ua + jnp.concatenate([z, ua[:, :_LW - sh, :]], axis=1)
            ub = ub + jnp.concatenate([z, ub[:, :_LW - sh, :]], axis=1)
        at = dnext * a[:, None, :] + dpow * ua
        bt = dnext * b[:, None, :] + dpow * ub
        wkvbuf[:, pl.ds(j * _LW, _LW), :] = at / (bt + 1e-8)
        return at[:, _LW - 1, :], bt[:, _LW - 1, :]

    a_fin, b_fin = lax.fori_loop(0, _SC_CHUNK // _LW, win,
                                 (a_ref[...], b_ref[...]))
    a_ref[...] = a_fin
    b_ref[...] = b_fin

    h2 = hm + r2 * wkvbuf[...].reshape(rows, dmodel)

    q = jnp.dot(h2, wq_ref[...], preferred_element_type=f32)
    scores = jnp.dot(q, mkt_ref[...], preferred_element_type=f32)
    scores = scores / jnp.sqrt(f32(dmodel))
    colid = lax.broadcasted_iota(jnp.int32, (rows, mkt_ref.shape[1]), 1)
    neg = f32(-1e30)
    scores = jnp.where(colid < cap, scores, neg)
    m1 = jnp.max(scores, axis=1, keepdims=True)
    i1 = jnp.min(jnp.where(scores == m1, colid, 9999), axis=1, keepdims=True)
    mask1 = colid == i1
    s2 = jnp.where(mask1, neg, scores)
    m2 = jnp.max(s2, axis=1, keepdims=True)
    i2 = jnp.min(jnp.where(s2 == m2, colid, 9999), axis=1, keepdims=True)
    mask2 = colid == i2
    e2 = jnp.exp(m2 - m1)
    inv = 1.0 / (1.0 + e2)
    wfull = jnp.where(mask1, inv, 0.0) + jnp.where(mask2, e2 * inv, 0.0)
    retrieved = jnp.dot(wfull, mv_ref[...], preferred_element_type=f32)
    h3 = h2 + jnp.dot(retrieved, wo_ref[...], preferred_element_type=f32)

    out = jnp.dot(h3, ow_ref[...], preferred_element_type=f32) + ob_ref[...]
    out_ref[...] = out.reshape(bsz, _SC_CHUNK, vocab)


def kernel(x, embed, time_decay, Wk, Wv, Wr, mem_keys, mem_vals, Wq, Wo,
           out_W, out_b):
    bsz, seq = x.shape
    vocab, dmodel = embed.shape
    cap = mem_keys.shape[0]
    capp = 128

    h = _embed_gather(x.reshape(-1).astype(jnp.int32), embed)
    h = h.reshape(bsz, seq, dmodel)

    mkt = jnp.zeros((dmodel, capp), jnp.float32).at[:, :cap].set(mem_keys.T)
    mv = jnp.zeros((capp, dmodel), jnp.float32).at[:cap].set(mem_vals)
    td2 = time_decay.reshape(1, dmodel)
    ob2 = out_b.reshape(1, vocab)

    grid = (seq // _SC_CHUNK,)
    out = pl.pallas_call(
        _tc_body,
        grid=grid,
        in_specs=[
            pl.BlockSpec((bsz, _SC_CHUNK, dmodel), lambda i: (0, i, 0)),
            pl.BlockSpec((1, dmodel), lambda i: (0, 0)),
            pl.BlockSpec((dmodel, dmodel), lambda i: (0, 0)),
            pl.BlockSpec((dmodel, dmodel), lambda i: (0, 0)),
            pl.BlockSpec((dmodel, dmodel), lambda i: (0, 0)),
            pl.BlockSpec((dmodel, capp), lambda i: (0, 0)),
            pl.BlockSpec((capp, dmodel), lambda i: (0, 0)),
            pl.BlockSpec((dmodel, dmodel), lambda i: (0, 0)),
            pl.BlockSpec((dmodel, dmodel), lambda i: (0, 0)),
            pl.BlockSpec((dmodel, vocab), lambda i: (0, 0)),
            pl.BlockSpec((1, vocab), lambda i: (0, 0)),
        ],
        out_specs=pl.BlockSpec((bsz, _SC_CHUNK, vocab), lambda i: (0, i, 0)),
        out_shape=jax.ShapeDtypeStruct((bsz, seq, vocab), jnp.float32),
        scratch_shapes=[
            pltpu.VMEM((bsz, _SC_CHUNK, dmodel), jnp.float32),
            pltpu.VMEM((bsz, _SC_CHUNK, dmodel), jnp.float32),
            pltpu.VMEM((bsz, _SC_CHUNK, dmodel), jnp.float32),
            pltpu.VMEM((bsz, dmodel), jnp.float32),
            pltpu.VMEM((bsz, dmodel), jnp.float32),
        ],
    )(h, td2, Wk, Wv, Wr, mkt, mv, Wq, Wo, out_W, ob2)
    return out


# speedup vs baseline: 51.9490x; 51.9490x over previous
import functools

import jax
import jax.numpy as jnp
from jax import lax
from jax.experimental import pallas as pl
from jax.experimental.pallas import tpu as pltpu
from jax.experimental.pallas import tpu_sc as plsc

_LW = 8
_SC_CHUNK = 256
_NW = 32


def _embed_gather(x_flat, embed):
    ntok = x_flat.shape[0]
    d = embed.shape[1]
    bpw = ntok // _NW
    mesh = plsc.VectorSubcoreMesh(core_axis_name="c", subcore_axis_name="s")

    @functools.partial(
        pl.kernel,
        mesh=mesh,
        out_type=jax.ShapeDtypeStruct((ntok, d), jnp.float32),
        scratch_types=[
            pltpu.VMEM((bpw,), jnp.int32),
            pltpu.VMEM((bpw, d), jnp.float32),
            pltpu.SemaphoreType.DMA,
        ],
    )
    def gather_kernel(table_hbm, idx_hbm, out_hbm, idx_v, rows_v, sem):
        wid = lax.axis_index("s") * 2 + lax.axis_index("c")
        base = wid * bpw
        pltpu.sync_copy(idx_hbm.at[pl.ds(base, bpw)], idx_v)
        pltpu.async_copy(table_hbm.at[idx_v], rows_v, sem).wait()
        pltpu.sync_copy(rows_v, out_hbm.at[pl.ds(base, bpw)])

    return gather_kernel(embed, x_flat)


def _tc_body(h_ref, td_ref, wk_ref, wv_ref, wr_ref, mkt_ref, mv_ref,
             wq_ref, wo_ref, ow_ref, ob_ref, out_ref,
             kbuf, vbuf, wkvbuf, a_ref, b_ref):
    bsz = h_ref.shape[0]
    dmodel = h_ref.shape[2]
    vocab = out_ref.shape[2]
    cap = 50
    rows = bsz * _SC_CHUNK
    f32 = jnp.float32

    @pl.when(pl.program_id(0) == 0)
    def _init():
        a_ref[...] = jnp.zeros_like(a_ref)
        b_ref[...] = jnp.zeros_like(b_ref)

    hm = h_ref[...].reshape(rows, dmodel)
    k2 = jnp.dot(hm, wk_ref[...], preferred_element_type=f32)
    v2 = jnp.dot(hm, wv_ref[...], preferred_element_type=f32)
    r2 = 1.0 / (1.0 + jnp.exp(-jnp.dot(hm, wr_ref[...], preferred_element_type=f32)))
    kbuf[...] = k2.reshape(bsz, _SC_CHUNK, dmodel)
    vbuf[...] = v2.reshape(bsz, _SC_CHUNK, dmodel)

    neg_ld = jnp.exp(td_ref[...])
    tpow = lax.broadcasted_iota(jnp.int32, (_LW, dmodel), 0).astype(f32)
    dpow = jnp.exp(-tpow * neg_ld)[None]
    dinv = jnp.exp(tpow * neg_ld)[None]
    dnext = dpow * jnp.exp(-neg_ld)[None]

    def win(j, carry):
        a, b = carry
        kk = kbuf[:, pl.ds(j * _LW, _LW), :]
        vv = vbuf[:, pl.ds(j * _LW, _LW), :]
        ek = jnp.exp(jnp.clip(kk, -30.0, 30.0))
        ua = ek * vv * dinv
        ub = ek * dinv
        for sh in (1, 2, 4):
            z = jnp.zeros((bsz, sh, dmodel), f32)
            ua = ua + jnp.concatenate([z, ua[:, :_LW - sh, :]], axis=1)
            ub = ub + jnp.concatenate([z, ub[:, :_LW - sh, :]], axis=1)
        at = dnext * a[:, None, :] + dpow * ua
        bt = dnext * b[:, None, :] + dpow * ub
        wkvbuf[:, pl.ds(j * _LW, _LW), :] = at / (bt + 1e-8)
        return at[:, _LW - 1, :], bt[:, _LW - 1, :]

    a_fin, b_fin = lax.fori_loop(0, _SC_CHUNK // _LW, win,
                                 (a_ref[...], b_ref[...]))
    a_ref[...] = a_fin
    b_ref[...] = b_fin

    h2 = hm + r2 * wkvbuf[...].reshape(rows, dmodel)

    q = jnp.dot(h2, wq_ref[...], preferred_element_type=f32)
    scores = jnp.dot(q, mkt_ref[...], preferred_element_type=f32)
    scores = scores / jnp.sqrt(f32(dmodel))
    colid = lax.broadcasted_iota(jnp.int32, (rows, mkt_ref.shape[1]), 1)
    neg = f32(-1e30)
    scores = jnp.where(colid < cap, scores, neg)
    m1 = jnp.max(scores, axis=1, keepdims=True)
    i1 = jnp.min(jnp.where(scores == m1, colid, 9999), axis=1, keepdims=True)
    mask1 = colid == i1
    s2 = jnp.where(mask1, neg, scores)
    m2 = jnp.max(s2, axis=1, keepdims=True)
    i2 = jnp.min(jnp.where(s2 == m2, colid, 9999), axis=1, keepdims=True)
    mask2 = colid == i2
    e2 = jnp.exp(m2 - m1)
    inv = 1.0 / (1.0 + e2)
    wfull = jnp.where(mask1, inv, 0.0) + jnp.where(mask2, e2 * inv, 0.0)
    retrieved = jnp.dot(wfull, mv_ref[...], preferred_element_type=f32)
    h3 = h2 + jnp.dot(retrieved, wo_ref[...], preferred_element_type=f32)

    out = jnp.dot(h3, ow_ref[...], preferred_element_type=f32) + ob_ref[...]
    out_ref[...] = out.reshape(bsz, _SC_CHUNK, vocab)


def kernel(x, embed, time_decay, Wk, Wv, Wr, mem_keys, mem_vals, Wq, Wo,
           out_W, out_b):
    bsz, seq = x.shape
    vocab, dmodel = embed.shape
    cap = mem_keys.shape[0]
    capp = 128

    h = _embed_gather(x.reshape(-1).astype(jnp.int32), embed)
    h = h.reshape(bsz, seq, dmodel)

    mkt = jnp.zeros((dmodel, capp), jnp.float32).at[:, :cap].set(mem_keys.T)
    mv = jnp.zeros((capp, dmodel), jnp.float32).at[:cap].set(mem_vals)
    td2 = time_decay.reshape(1, dmodel)
    ob2 = out_b.reshape(1, vocab)

    grid = (seq // _SC_CHUNK,)
    out = pl.pallas_call(
        _tc_body,
        grid=grid,
        in_specs=[
            pl.BlockSpec((bsz, _SC_CHUNK, dmodel), lambda i: (0, i, 0)),
            pl.BlockSpec((1, dmodel), lambda i: (0, 0)),
            pl.BlockSpec((dmodel, dmodel), lambda i: (0, 0)),
            pl.BlockSpec((dmodel, dmodel), lambda i: (0, 0)),
            pl.BlockSpec((dmodel, dmodel), lambda i: (0, 0)),
            pl.BlockSpec((dmodel, capp), lambda i: (0, 0)),
            pl.BlockSpec((capp, dmodel), lambda i: (0, 0)),
            pl.BlockSpec((dmodel, dmodel), lambda i: (0, 0)),
            pl.BlockSpec((dmodel, dmodel), lambda i: (0, 0)),
            pl.BlockSpec((dmodel, vocab), lambda i: (0, 0)),
            pl.BlockSpec((1, vocab), lambda i: (0, 0)),
        ],
        out_specs=pl.BlockSpec((bsz, _SC_CHUNK, vocab), lambda i: (0, i, 0)),
        out_shape=jax.ShapeDtypeStruct((bsz, seq, vocab), jnp.float32),
        scratch_shapes=[
            pltpu.VMEM((bsz, _SC_CHUNK, dmodel), jnp.float32),
            pltpu.VMEM((bsz, _SC_CHUNK, dmodel), jnp.float32),
            pltpu.VMEM((bsz, _SC_CHUNK, dmodel), jnp.float32),
            pltpu.VMEM((bsz, dmodel), jnp.float32),
            pltpu.VMEM((bsz, dmodel), jnp.float32),
        ],
    )(h, td2, Wk, Wv, Wr, mkt, mv, Wq, Wo, out_W, ob2)
    return out


# speedup vs baseline: 53.8652x; 1.0369x over previous
import functools

import jax
import jax.numpy as jnp
from jax import lax
from jax.experimental import pallas as pl
from jax.experimental.pallas import tpu as pltpu
from jax.experimental.pallas import tpu_sc as plsc

_LW = 8
_SC_CHUNK = 256
_NW = 32


def _embed_gather(x_flat, embed):
    ntok = x_flat.shape[0]
    d = embed.shape[1]
    bpw = ntok // _NW
    mesh = plsc.VectorSubcoreMesh(core_axis_name="c", subcore_axis_name="s")

    @functools.partial(
        pl.kernel,
        mesh=mesh,
        out_type=jax.ShapeDtypeStruct((ntok, d), jnp.float32),
        scratch_types=[
            pltpu.VMEM((bpw,), jnp.int32),
            pltpu.VMEM((bpw, d), jnp.float32),
            pltpu.SemaphoreType.DMA,
        ],
    )
    def gather_kernel(table_hbm, idx_hbm, out_hbm, idx_v, rows_v, sem):
        wid = lax.axis_index("s") * 2 + lax.axis_index("c")
        base = wid * bpw
        pltpu.sync_copy(idx_hbm.at[pl.ds(base, bpw)], idx_v)
        pltpu.async_copy(table_hbm.at[idx_v], rows_v, sem).wait()
        pltpu.sync_copy(rows_v, out_hbm.at[pl.ds(base, bpw)])

    return gather_kernel(embed, x_flat)


def _tc_body(h_ref, td_ref, wk_ref, wv_ref, wr_ref, mkt_ref, mv_ref,
             wq_ref, wo_ref, ow_ref, ob_ref, out_ref,
             dm1_ref, dm2_ref, dm4_ref, dnx_ref, a_ref, b_ref):
    bsz = h_ref.shape[0]
    dmodel = h_ref.shape[2]
    vocab = out_ref.shape[2]
    cap = 50
    rows = bsz * _SC_CHUNK
    nw = _SC_CHUNK // _LW
    f32 = jnp.float32

    ne = jnp.exp(td_ref[...])

    @pl.when(pl.program_id(0) == 0)
    def _init():
        a_ref[...] = jnp.zeros_like(a_ref)
        b_ref[...] = jnp.zeros_like(b_ref)
        rowmod = (lax.broadcasted_iota(jnp.int32, (rows, dmodel), 0) & (_LW - 1)
                  ).astype(f32)
        dm1_ref[...] = jnp.where(rowmod >= 1, jnp.exp(-1.0 * ne), 0.0)
        dm2_ref[...] = jnp.where(rowmod >= 2, jnp.exp(-2.0 * ne), 0.0)
        dm4_ref[...] = jnp.where(rowmod >= 4, jnp.exp(-4.0 * ne), 0.0)
        dnx_ref[...] = jnp.exp(-(rowmod + 1.0) * ne)

    hm = h_ref[...].reshape(rows, dmodel)
    k2 = jnp.dot(hm, wk_ref[...], preferred_element_type=f32)
    v2 = jnp.dot(hm, wv_ref[...], preferred_element_type=f32)
    r2 = 1.0 / (1.0 + jnp.exp(-jnp.dot(hm, wr_ref[...], preferred_element_type=f32)))

    ek = jnp.exp(jnp.clip(k2, -30.0, 30.0))
    ua = ek * v2
    ub = ek
    for dm_ref, sh in ((dm1_ref, 1), (dm2_ref, 2), (dm4_ref, 4)):
        dm = dm_ref[...]
        z = jnp.zeros((sh, dmodel), f32)
        ua = ua + dm * jnp.concatenate([z, ua[:rows - sh, :]], axis=0)
        ub = ub + dm * jnp.concatenate([z, ub[:rows - sh, :]], axis=0)

    ta = ua.reshape(rows // _LW, _LW, dmodel)[:, _LW - 1, :]
    tb = ub.reshape(rows // _LW, _LW, dmodel)[:, _LW - 1, :]
    ta = ta.reshape(bsz, nw, dmodel)
    tb = tb.reshape(bsz, nw, dmodel)
    sh = 1
    while sh < nw:
        dsh = jnp.exp(f32(-_LW * sh) * ne)
        z = jnp.zeros((bsz, sh, dmodel), f32)
        ta = ta + dsh * jnp.concatenate([z, ta[:, :nw - sh, :]], axis=1)
        tb = tb + dsh * jnp.concatenate([z, tb[:, :nw - sh, :]], axis=1)
        sh *= 2
    jpos = (lax.broadcasted_iota(jnp.int32, (nw, dmodel), 0) + 1).astype(f32)
    p8 = jnp.exp((-float(_LW) * jpos) * ne)[None]
    a_in = a_ref[...]
    b_in = b_ref[...]
    afull = p8 * a_in[:, None, :] + ta
    bfull = p8 * b_in[:, None, :] + tb
    a_ref[...] = afull[:, nw - 1, :]
    b_ref[...] = bfull[:, nw - 1, :]
    aprev = jnp.concatenate([a_in[:, None, :], afull[:, :nw - 1, :]], axis=1)
    bprev = jnp.concatenate([b_in[:, None, :], bfull[:, :nw - 1, :]], axis=1)
    aex = jnp.broadcast_to(aprev.reshape(rows // _LW, 1, dmodel),
                           (rows // _LW, _LW, dmodel)).reshape(rows, dmodel)
    bex = jnp.broadcast_to(bprev.reshape(rows // _LW, 1, dmodel),
                           (rows // _LW, _LW, dmodel)).reshape(rows, dmodel)
    dnx = dnx_ref[...]
    wkv = (dnx * aex + ua) / (dnx * bex + ub + 1e-8)
    h2 = hm + r2 * wkv

    q = jnp.dot(h2, wq_ref[...], preferred_element_type=f32)
    scores = jnp.dot(q, mkt_ref[...], preferred_element_type=f32)
    scores = scores / jnp.sqrt(f32(dmodel))
    colid = lax.broadcasted_iota(jnp.int32, (rows, mkt_ref.shape[1]), 1
                                 ).astype(f32)
    neg = f32(-1e30)
    scores = jnp.where(colid < cap, scores, neg)
    m1 = jnp.max(scores, axis=1, keepdims=True)
    i1 = jnp.min(jnp.where(scores == m1, colid, f32(1e9)), axis=1, keepdims=True)
    mask1 = colid == i1
    s2 = jnp.where(mask1, neg, scores)
    m2 = jnp.max(s2, axis=1, keepdims=True)
    i2 = jnp.min(jnp.where(s2 == m2, colid, f32(1e9)), axis=1, keepdims=True)
    mask2 = colid == i2
    e2 = jnp.exp(m2 - m1)
    inv = 1.0 / (1.0 + e2)
    wfull = jnp.where(mask1, inv, 0.0) + jnp.where(mask2, e2 * inv, 0.0)
    retrieved = jnp.dot(wfull, mv_ref[...], preferred_element_type=f32)
    h3 = h2 + jnp.dot(retrieved, wo_ref[...], preferred_element_type=f32)

    out = jnp.dot(h3, ow_ref[...], preferred_element_type=f32) + ob_ref[...]
    out_ref[...] = out.reshape(bsz, _SC_CHUNK, vocab)


def kernel(x, embed, time_decay, Wk, Wv, Wr, mem_keys, mem_vals, Wq, Wo,
           out_W, out_b):
    bsz, seq = x.shape
    vocab, dmodel = embed.shape
    cap = mem_keys.shape[0]
    capp = 128

    h = _embed_gather(x.reshape(-1).astype(jnp.int32), embed)
    h = h.reshape(bsz, seq, dmodel)

    mkt = jnp.zeros((dmodel, capp), jnp.float32).at[:, :cap].set(mem_keys.T)
    mv = jnp.zeros((capp, dmodel), jnp.float32).at[:cap].set(mem_vals)
    td2 = time_decay.reshape(1, dmodel)
    ob2 = out_b.reshape(1, vocab)

    grid = (seq // _SC_CHUNK,)
    out = pl.pallas_call(
        _tc_body,
        grid=grid,
        in_specs=[
            pl.BlockSpec((bsz, _SC_CHUNK, dmodel), lambda i: (0, i, 0)),
            pl.BlockSpec((1, dmodel), lambda i: (0, 0)),
            pl.BlockSpec((dmodel, dmodel), lambda i: (0, 0)),
            pl.BlockSpec((dmodel, dmodel), lambda i: (0, 0)),
            pl.BlockSpec((dmodel, dmodel), lambda i: (0, 0)),
            pl.BlockSpec((dmodel, capp), lambda i: (0, 0)),
            pl.BlockSpec((capp, dmodel), lambda i: (0, 0)),
            pl.BlockSpec((dmodel, dmodel), lambda i: (0, 0)),
            pl.BlockSpec((dmodel, dmodel), lambda i: (0, 0)),
            pl.BlockSpec((dmodel, vocab), lambda i: (0, 0)),
            pl.BlockSpec((1, vocab), lambda i: (0, 0)),
        ],
        out_specs=pl.BlockSpec((bsz, _SC_CHUNK, vocab), lambda i: (0, i, 0)),
        out_shape=jax.ShapeDtypeStruct((bsz, seq, vocab), jnp.float32),
        scratch_shapes=[
            pltpu.VMEM((bsz * _SC_CHUNK, dmodel), jnp.float32),
            pltpu.VMEM((bsz * _SC_CHUNK, dmodel), jnp.float32),
            pltpu.VMEM((bsz * _SC_CHUNK, dmodel), jnp.float32),
            pltpu.VMEM((bsz * _SC_CHUNK, dmodel), jnp.float32),
            pltpu.VMEM((bsz, dmodel), jnp.float32),
            pltpu.VMEM((bsz, dmodel), jnp.float32),
        ],
    )(h, td2, Wk, Wv, Wr, mkt, mv, Wq, Wo, out_W, ob2)
    return out


# speedup vs baseline: 86.0406x; 1.5973x over previous
import functools

import jax
import jax.numpy as jnp
from jax import lax
from jax.experimental import pallas as pl
from jax.experimental.pallas import tpu as pltpu
from jax.experimental.pallas import tpu_sc as plsc

_LW = 8
_SC_CHUNK = 256
_NW = 32


def _embed_gather(x_flat, embed):
    ntok = x_flat.shape[0]
    d = embed.shape[1]
    bpw = ntok // _NW
    mesh = plsc.VectorSubcoreMesh(core_axis_name="c", subcore_axis_name="s")

    @functools.partial(
        pl.kernel,
        mesh=mesh,
        out_type=jax.ShapeDtypeStruct((ntok, d), jnp.float32),
        scratch_types=[
            pltpu.VMEM((bpw,), jnp.int32),
            pltpu.VMEM((bpw, d), jnp.float32),
            pltpu.SemaphoreType.DMA,
        ],
    )
    def gather_kernel(table_hbm, idx_hbm, out_hbm, idx_v, rows_v, sem):
        wid = lax.axis_index("s") * 2 + lax.axis_index("c")
        base = wid * bpw
        pltpu.sync_copy(idx_hbm.at[pl.ds(base, bpw)], idx_v)
        pltpu.async_copy(table_hbm.at[idx_v], rows_v, sem).wait()
        pltpu.sync_copy(rows_v, out_hbm.at[pl.ds(base, bpw)])

    return gather_kernel(embed, x_flat)


def _tc_body(h_ref, td_ref, wk_ref, wv_ref, wr_ref, mkt_ref, mv_ref,
             wq_ref, wo_ref, ow_ref, ob_ref, out_ref,
             dm1_ref, dm2_ref, dm4_ref, dnx_ref, a_ref, b_ref):
    bsz = h_ref.shape[0]
    dmodel = h_ref.shape[2]
    vocab = out_ref.shape[1]
    cap = 50
    rows = bsz * _SC_CHUNK
    nw = _SC_CHUNK // _LW
    f32 = jnp.float32

    ne = jnp.exp(td_ref[...])

    @pl.when(pl.program_id(0) == 0)
    def _init():
        a_ref[...] = jnp.zeros_like(a_ref)
        b_ref[...] = jnp.zeros_like(b_ref)
        rowmod = (lax.broadcasted_iota(jnp.int32, (rows, dmodel), 0) & (_LW - 1)
                  ).astype(f32)
        dm1_ref[...] = jnp.where(rowmod >= 1, jnp.exp(-1.0 * ne), 0.0)
        dm2_ref[...] = jnp.where(rowmod >= 2, jnp.exp(-2.0 * ne), 0.0)
        dm4_ref[...] = jnp.where(rowmod >= 4, jnp.exp(-4.0 * ne), 0.0)
        dnx_ref[...] = jnp.exp(-(rowmod + 1.0) * ne)

    hm = h_ref[...].reshape(rows, dmodel)
    k2 = jnp.dot(hm, wk_ref[...], preferred_element_type=f32)
    v2 = jnp.dot(hm, wv_ref[...], preferred_element_type=f32)
    r2 = 1.0 / (1.0 + jnp.exp(-jnp.dot(hm, wr_ref[...], preferred_element_type=f32)))

    ek = jnp.exp(jnp.clip(k2, -30.0, 30.0))
    ua = ek * v2
    ub = ek
    for dm_ref, sh in ((dm1_ref, 1), (dm2_ref, 2), (dm4_ref, 4)):
        dm = dm_ref[...]
        z = jnp.zeros((sh, dmodel), f32)
        ua = ua + dm * jnp.concatenate([z, ua[:rows - sh, :]], axis=0)
        ub = ub + dm * jnp.concatenate([z, ub[:rows - sh, :]], axis=0)

    ta = ua.reshape(rows // _LW, _LW, dmodel)[:, _LW - 1, :]
    tb = ub.reshape(rows // _LW, _LW, dmodel)[:, _LW - 1, :]
    ta = ta.reshape(bsz, nw, dmodel)
    tb = tb.reshape(bsz, nw, dmodel)
    sh = 1
    while sh < nw:
        dsh = jnp.exp(f32(-_LW * sh) * ne)
        z = jnp.zeros((bsz, sh, dmodel), f32)
        ta = ta + dsh * jnp.concatenate([z, ta[:, :nw - sh, :]], axis=1)
        tb = tb + dsh * jnp.concatenate([z, tb[:, :nw - sh, :]], axis=1)
        sh *= 2
    jpos = (lax.broadcasted_iota(jnp.int32, (nw, dmodel), 0) + 1).astype(f32)
    p8 = jnp.exp((-float(_LW) * jpos) * ne)[None]
    a_in = a_ref[...]
    b_in = b_ref[...]
    afull = p8 * a_in[:, None, :] + ta
    bfull = p8 * b_in[:, None, :] + tb
    a_ref[...] = afull[:, nw - 1, :]
    b_ref[...] = bfull[:, nw - 1, :]
    aprev = jnp.concatenate([a_in[:, None, :], afull[:, :nw - 1, :]], axis=1)
    bprev = jnp.concatenate([b_in[:, None, :], bfull[:, :nw - 1, :]], axis=1)
    aex = jnp.broadcast_to(aprev.reshape(rows // _LW, 1, dmodel),
                           (rows // _LW, _LW, dmodel)).reshape(rows, dmodel)
    bex = jnp.broadcast_to(bprev.reshape(rows // _LW, 1, dmodel),
                           (rows // _LW, _LW, dmodel)).reshape(rows, dmodel)
    dnx = dnx_ref[...]
    wkv = (dnx * aex + ua) / (dnx * bex + ub + 1e-8)
    h2 = hm + r2 * wkv

    q = jnp.dot(h2, wq_ref[...], preferred_element_type=f32)
    scores = jnp.dot(q, mkt_ref[...], preferred_element_type=f32)
    scores = scores / jnp.sqrt(f32(dmodel))
    colid = lax.broadcasted_iota(jnp.int32, (rows, mkt_ref.shape[1]), 1
                                 ).astype(f32)
    neg = f32(-1e30)
    scores = jnp.where(colid < cap, scores, neg)
    m1 = jnp.max(scores, axis=1, keepdims=True)
    i1 = jnp.min(jnp.where(scores == m1, colid, f32(1e9)), axis=1, keepdims=True)
    mask1 = colid == i1
    s2 = jnp.where(mask1, neg, scores)
    m2 = jnp.max(s2, axis=1, keepdims=True)
    i2 = jnp.min(jnp.where(s2 == m2, colid, f32(1e9)), axis=1, keepdims=True)
    mask2 = colid == i2
    e2 = jnp.exp(m2 - m1)
    inv = 1.0 / (1.0 + e2)
    wfull = jnp.where(mask1, inv, 0.0) + jnp.where(mask2, e2 * inv, 0.0)
    retrieved = jnp.dot(wfull, mv_ref[...], preferred_element_type=f32)
    h3 = h2 + jnp.dot(retrieved, wo_ref[...], preferred_element_type=f32)

    owt = ow_ref[...]
    ob = ob_ref[...]
    for b in range(bsz):
        h3b = h3[b * _SC_CHUNK:(b + 1) * _SC_CHUNK, :]
        out_ref[b] = jnp.dot(owt, h3b.T, preferred_element_type=f32) + ob


def kernel(x, embed, time_decay, Wk, Wv, Wr, mem_keys, mem_vals, Wq, Wo,
           out_W, out_b):
    bsz, seq = x.shape
    vocab, dmodel = embed.shape
    cap = mem_keys.shape[0]
    capp = 128

    h = _embed_gather(x.reshape(-1).astype(jnp.int32), embed)
    h = h.reshape(bsz, seq, dmodel)

    mkt = jnp.zeros((dmodel, capp), jnp.float32).at[:, :cap].set(mem_keys.T)
    mv = jnp.zeros((capp, dmodel), jnp.float32).at[:cap].set(mem_vals)
    td2 = time_decay.reshape(1, dmodel)
    owt = out_W.T
    obc = out_b.reshape(vocab, 1)

    grid = (seq // _SC_CHUNK,)
    out = pl.pallas_call(
        _tc_body,
        grid=grid,
        in_specs=[
            pl.BlockSpec((bsz, _SC_CHUNK, dmodel), lambda i: (0, i, 0)),
            pl.BlockSpec((1, dmodel), lambda i: (0, 0)),
            pl.BlockSpec((dmodel, dmodel), lambda i: (0, 0)),
            pl.BlockSpec((dmodel, dmodel), lambda i: (0, 0)),
            pl.BlockSpec((dmodel, dmodel), lambda i: (0, 0)),
            pl.BlockSpec((dmodel, capp), lambda i: (0, 0)),
            pl.BlockSpec((capp, dmodel), lambda i: (0, 0)),
            pl.BlockSpec((dmodel, dmodel), lambda i: (0, 0)),
            pl.BlockSpec((dmodel, dmodel), lambda i: (0, 0)),
            pl.BlockSpec((vocab, dmodel), lambda i: (0, 0)),
            pl.BlockSpec((vocab, 1), lambda i: (0, 0)),
        ],
        out_specs=pl.BlockSpec((bsz, vocab, _SC_CHUNK), lambda i: (0, 0, i)),
        out_shape=jax.ShapeDtypeStruct((bsz, vocab, seq), jnp.float32),
        scratch_shapes=[
            pltpu.VMEM((bsz * _SC_CHUNK, dmodel), jnp.float32),
            pltpu.VMEM((bsz * _SC_CHUNK, dmodel), jnp.float32),
            pltpu.VMEM((bsz * _SC_CHUNK, dmodel), jnp.float32),
            pltpu.VMEM((bsz * _SC_CHUNK, dmodel), jnp.float32),
            pltpu.VMEM((bsz, dmodel), jnp.float32),
            pltpu.VMEM((bsz, dmodel), jnp.float32),
        ],
    )(h, td2, Wk, Wv, Wr, mkt, mv, Wq, Wo, owt, obc)
    return jnp.swapaxes(out, 1, 2)
